# Initial kernel scaffold; baseline (speedup 1.0000x reference)
import functools

import jax
import jax.numpy as jnp
from jax import lax
from jax.experimental import pallas as pl
from jax.experimental.pallas import tpu as pltpu
from jax.experimental.pallas import tpu_sc as plsc

N = 10000
E = 320000
D = 128

NC = 2
NS = 16
NW = NC * NS
EPW = E // NW
C = 80
NCHUNK = EPW // C
RPT = N // NS
NLANE = D // 16


def _matmul_body(x_ref, w_ref, b_ref, o_ref):
    o_ref[...] = (
        jnp.dot(x_ref[...], w_ref[...], preferred_element_type=jnp.float32)
        + b_ref[...]
    )


def _node_matmul(node_feats, w_cat, b_cat):
    return pl.pallas_call(
        _matmul_body,
        out_shape=jax.ShapeDtypeStruct((N, 4 * D), jnp.float32),
    )(node_feats, w_cat, b_cat)


_EDGE_BLK = 1280


def _edge_matmul(edge_feats, w, b):
    grid = (E // _EDGE_BLK,)
    return pl.pallas_call(
        _matmul_body,
        grid=grid,
        in_specs=[
            pl.BlockSpec((_EDGE_BLK, D), lambda i: (i, 0)),
            pl.BlockSpec((D, D), lambda i: (0, 0)),
            pl.BlockSpec((1, D), lambda i: (0, 0)),
        ],
        out_specs=pl.BlockSpec((_EDGE_BLK, D), lambda i: (i, 0)),
        out_shape=jax.ShapeDtypeStruct((E, D), jnp.float32),
    )(edge_feats, w, b)


_SC_MESH = plsc.VectorSubcoreMesh(core_axis_name="c", subcore_axis_name="s")


@functools.partial(
    pl.kernel,
    out_type=(
        jax.ShapeDtypeStruct((E, D), jnp.float32),
        jax.ShapeDtypeStruct((NC, N, D), jnp.float32),
        jax.ShapeDtypeStruct((NW, 2, D), jnp.float32),
    ),
    mesh=_SC_MESH,
    scratch_types=[
        pltpu.VMEM((C,), jnp.int32),
        pltpu.VMEM((C,), jnp.int32),
        pltpu.VMEM((C, D), jnp.float32),
        pltpu.VMEM((C, D), jnp.float32),
        pltpu.VMEM((C, D), jnp.float32),
        pltpu.VMEM((2, D), jnp.float32),
        pltpu.VMEM_SHARED((N, D), jnp.float32),
        pltpu.SemaphoreType.DMA,
        pltpu.SemaphoreType.DMA,
    ],
)
def _sc_edge_kernel(lin_hbm, src_hbm, dst_hbm, esrc_hbm, edst_hbm, zeros_hbm,
                    upd_hbm, gsum_hbm, stats_hbm,
                    isrc, idst, linb, gsrc, gdst, statsb, acc, sem0, sem1):
    cid = lax.axis_index("c")
    sid = lax.axis_index("s")
    wid = sid * NC + cid

    pltpu.sync_copy(
        zeros_hbm.at[pl.ds(sid * RPT, RPT), :],
        acc.at[pl.ds(sid * RPT, RPT), :],
    )
    plsc.subcore_barrier()

    def chunk_body(k, carry):
        base = wid * EPW + k * C
        pltpu.sync_copy(src_hbm.at[pl.ds(base, C)], isrc)
        pltpu.sync_copy(dst_hbm.at[pl.ds(base, C)], idst)
        cp0 = pltpu.async_copy(esrc_hbm.at[isrc], gsrc, sem0)
        cp1 = pltpu.async_copy(edst_hbm.at[idst], gdst, sem1)
        pltpu.sync_copy(lin_hbm.at[pl.ds(base, C), :], linb)
        cp0.wait()
        cp1.wait()

        def row_body(r, rc):
            sums, sqs = rc
            new_sums, new_sqs = [], []
            for j in range(NLANE):
                sl = pl.ds(j * 16, 16)
                u = linb[r, sl] + gsrc[r, sl] + gdst[r, sl]
                linb[r, sl] = u
                gsrc[r, sl] = u / (1.0 + jnp.exp(-u))
                new_sums.append(sums[j] + u)
                new_sqs.append(sqs[j] + u * u)
            return (tuple(new_sums), tuple(new_sqs))

        carry = lax.fori_loop(0, C, row_body, carry)
        pltpu.sync_copy(linb, upd_hbm.at[pl.ds(base, C), :])
        pltpu.sync_copy(gsrc, acc.at[idst], add=True)
        return carry

    z = jnp.zeros((16,), jnp.float32)
    init = (tuple(z for _ in range(NLANE)), tuple(z for _ in range(NLANE)))
    sums, sqs = lax.fori_loop(0, NCHUNK, chunk_body, init)

    for j in range(NLANE):
        statsb[0, pl.ds(j * 16, 16)] = sums[j]
        statsb[1, pl.ds(j * 16, 16)] = sqs[j]
    pltpu.sync_copy(statsb, stats_hbm.at[wid])

    plsc.subcore_barrier()
    pltpu.sync_copy(
        acc.at[pl.ds(sid * RPT, RPT), :],
        gsum_hbm.at[cid, pl.ds(sid * RPT, RPT), :],
    )


def _edge_out_body(upd_ref, ef_ref, stats_ref, gamma_ref, beta_ref, o_ref):
    s = jnp.sum(stats_ref[...], axis=0)
    mean = s[0] / E
    var = s[1] / E - mean * mean
    rstd = lax.rsqrt(var + 1e-5)
    u = upd_ref[...]
    xh = (u - mean[None, :]) * rstd[None, :] * gamma_ref[...] + beta_ref[...]
    o_ref[...] = xh * jax.nn.sigmoid(xh) + ef_ref[...]


def _edge_out(upd, edge_feats, stats, gamma, beta):
    grid = (E // _EDGE_BLK,)
    return pl.pallas_call(
        _edge_out_body,
        grid=grid,
        in_specs=[
            pl.BlockSpec((_EDGE_BLK, D), lambda i: (i, 0)),
            pl.BlockSpec((_EDGE_BLK, D), lambda i: (i, 0)),
            pl.BlockSpec((NW, 2, D), lambda i: (0, 0, 0)),
            pl.BlockSpec((1, D), lambda i: (0, 0)),
            pl.BlockSpec((1, D), lambda i: (0, 0)),
        ],
        out_specs=pl.BlockSpec((_EDGE_BLK, D), lambda i: (i, 0)),
        out_shape=jax.ShapeDtypeStruct((E, D), jnp.float32),
    )(upd, edge_feats, stats, gamma, beta)


def _node_out_body(gsum_ref, ndst_ref, nsrc_ref, nf_ref, gamma_ref, beta_ref,
                   o_ref):
    s = gsum_ref[0] + gsum_ref[1]
    n_gate = ndst_ref[...] * (s / (s + 1e-6))
    x = nsrc_ref[...] + n_gate
    mean = jnp.mean(x, axis=0, keepdims=True)
    var = jnp.mean((x - mean) ** 2, axis=0, keepdims=True)
    xh = (x - mean) * lax.rsqrt(var + 1e-5) * gamma_ref[...] + beta_ref[...]
    o_ref[...] = xh * jax.nn.sigmoid(xh) + nf_ref[...]


def _node_out(gsum, n_dst_lin, n_src_lin, node_feats, gamma, beta):
    return pl.pallas_call(
        _node_out_body,
        out_shape=jax.ShapeDtypeStruct((N, D), jnp.float32),
    )(gsum, n_dst_lin, n_src_lin, node_feats, gamma, beta)


def kernel(node_feats, edge_feats, W_esrc, b_esrc, W_edst, b_edst, W_eedge,
           b_eedge, W_nsrc, b_nsrc, W_ndst, b_ndst, gamma_e, beta_e, gamma_n,
           beta_n, edge_index):
    w_cat = jnp.concatenate([W_esrc, W_edst, W_ndst, W_nsrc], axis=1)
    b_cat = jnp.concatenate([b_esrc, b_edst, b_ndst, b_nsrc])[None, :]
    nodecat = _node_matmul(node_feats, w_cat, b_cat)
    e_src_t = nodecat[:, 0 * D:1 * D]
    e_dst_t = nodecat[:, 1 * D:2 * D]
    n_dst_lin = nodecat[:, 2 * D:3 * D]
    n_src_lin = nodecat[:, 3 * D:4 * D]

    edge_lin = _edge_matmul(edge_feats, W_eedge, b_eedge[None, :])

    src = edge_index[0]
    dst = edge_index[1]
    zeros = jnp.zeros((N, D), jnp.float32)
    upd, gsum, stats = _sc_edge_kernel(
        edge_lin, src, dst, e_src_t, e_dst_t, zeros)

    edge_out = _edge_out(upd, edge_feats, stats, gamma_e[None, :],
                         beta_e[None, :])
    node_out = _node_out(gsum, n_dst_lin, n_src_lin, node_feats,
                         gamma_n[None, :], beta_n[None, :])
    return (node_out, edge_out)


# speedup vs baseline: 3.6542x; 3.6542x over previous
import functools

import jax
import jax.numpy as jnp
from jax import lax
from jax.experimental import pallas as pl
from jax.experimental.pallas import tpu as pltpu
from jax.experimental.pallas import tpu_sc as plsc

N = 10000
E = 320000
D = 128

NC = 2
NS = 16
NW = NC * NS
EPW = E // NW
C = 80
NCHUNK = EPW // C
NPAD = 10240
RPT = NPAD // NS
NLANE = D // 16


def _matmul_body(x_ref, w_ref, b_ref, o_ref):
    o_ref[...] = (
        jnp.dot(x_ref[...], w_ref[...], preferred_element_type=jnp.float32)
        + b_ref[...]
    )


def _node_matmul(node_feats, w_cat, b_cat):
    return pl.pallas_call(
        _matmul_body,
        out_shape=jax.ShapeDtypeStruct((N, 4 * D), jnp.float32),
    )(node_feats, w_cat, b_cat)


_EDGE_BLK = 1280


def _edge_matmul(edge_feats, w, b):
    grid = (E // _EDGE_BLK,)
    return pl.pallas_call(
        _matmul_body,
        grid=grid,
        in_specs=[
            pl.BlockSpec((_EDGE_BLK, D), lambda i: (i, 0)),
            pl.BlockSpec((D, D), lambda i: (0, 0)),
            pl.BlockSpec((1, D), lambda i: (0, 0)),
        ],
        out_specs=pl.BlockSpec((_EDGE_BLK, D), lambda i: (i, 0)),
        out_shape=jax.ShapeDtypeStruct((E, D), jnp.float32),
    )(edge_feats, w, b)


_SC_MESH = plsc.VectorSubcoreMesh(core_axis_name="c", subcore_axis_name="s")


@functools.partial(
    pl.kernel,
    out_type=(
        jax.ShapeDtypeStruct((E, D), jnp.float32),
        jax.ShapeDtypeStruct((NC, NPAD, D), jnp.float32),
        jax.ShapeDtypeStruct((NW, 2, D), jnp.float32),
    ),
    mesh=_SC_MESH,
    scratch_types=[
        pltpu.VMEM((C,), jnp.int32),
        pltpu.VMEM((C,), jnp.int32),
        pltpu.VMEM((C, D), jnp.float32),
        pltpu.VMEM((C, D), jnp.float32),
        pltpu.VMEM((C, D), jnp.float32),
        pltpu.VMEM((2, D), jnp.float32),
        pltpu.VMEM_SHARED((NPAD, D), jnp.float32),
        pltpu.SemaphoreType.DMA,
        pltpu.SemaphoreType.DMA,
    ],
)
def _sc_edge_kernel(lin_hbm, src_hbm, dst_hbm, esrc_hbm, edst_hbm, zeros_hbm,
                    upd_hbm, gsum_hbm, stats_hbm,
                    isrc, idst, linb, gsrc, gdst, statsb, acc, sem0, sem1):
    cid = lax.axis_index("c")
    sid = lax.axis_index("s")
    wid = sid * NC + cid

    pltpu.sync_copy(
        zeros_hbm.at[pl.ds(sid * RPT, RPT), :],
        acc.at[pl.ds(sid * RPT, RPT), :],
    )
    plsc.subcore_barrier()

    def chunk_body(k, carry):
        base = wid * EPW + k * C
        pltpu.sync_copy(src_hbm.at[pl.ds(base, C)], isrc)
        pltpu.sync_copy(dst_hbm.at[pl.ds(base, C)], idst)
        cp0 = pltpu.async_copy(esrc_hbm.at[isrc], gsrc, sem0)
        cp1 = pltpu.async_copy(edst_hbm.at[idst], gdst, sem1)
        pltpu.sync_copy(lin_hbm.at[pl.ds(base, C), :], linb)
        cp0.wait()
        cp1.wait()

        def row_body(r, rc):
            sums, sqs = rc
            new_sums, new_sqs = [], []
            for j in range(NLANE):
                sl = pl.ds(j * 16, 16)
                u = linb[r, sl] + gsrc[r, sl] + gdst[r, sl]
                linb[r, sl] = u
                gsrc[r, sl] = u / (1.0 + jnp.exp(-u))
                new_sums.append(sums[j] + u)
                new_sqs.append(sqs[j] + u * u)
            return (tuple(new_sums), tuple(new_sqs))

        carry = lax.fori_loop(0, C, row_body, carry)
        pltpu.sync_copy(linb, upd_hbm.at[pl.ds(base, C), :])
        pltpu.sync_copy(gsrc, acc.at[idst], add=True)
        return carry

    z = jnp.zeros((16,), jnp.float32)
    init = (tuple(z for _ in range(NLANE)), tuple(z for _ in range(NLANE)))
    sums, sqs = lax.fori_loop(0, NCHUNK, chunk_body, init)

    for j in range(NLANE):
        statsb[0, pl.ds(j * 16, 16)] = sums[j]
        statsb[1, pl.ds(j * 16, 16)] = sqs[j]
    pltpu.sync_copy(statsb, stats_hbm.at[wid])

    plsc.subcore_barrier()
    pltpu.sync_copy(
        acc.at[pl.ds(sid * RPT, RPT), :],
        gsum_hbm.at[cid, pl.ds(sid * RPT, RPT), :],
    )


def _edge_out_body(upd_ref, ef_ref, stats_ref, gamma_ref, beta_ref, o_ref):
    s = jnp.sum(stats_ref[...], axis=0)
    mean = s[0] / E
    var = s[1] / E - mean * mean
    rstd = lax.rsqrt(var + 1e-5)
    u = upd_ref[...]
    xh = (u - mean[None, :]) * rstd[None, :] * gamma_ref[...] + beta_ref[...]
    o_ref[...] = xh * jax.nn.sigmoid(xh) + ef_ref[...]


def _edge_out(upd, edge_feats, stats, gamma, beta):
    grid = (E // _EDGE_BLK,)
    return pl.pallas_call(
        _edge_out_body,
        grid=grid,
        in_specs=[
            pl.BlockSpec((_EDGE_BLK, D), lambda i: (i, 0)),
            pl.BlockSpec((_EDGE_BLK, D), lambda i: (i, 0)),
            pl.BlockSpec((NW, 2, D), lambda i: (0, 0, 0)),
            pl.BlockSpec((1, D), lambda i: (0, 0)),
            pl.BlockSpec((1, D), lambda i: (0, 0)),
        ],
        out_specs=pl.BlockSpec((_EDGE_BLK, D), lambda i: (i, 0)),
        out_shape=jax.ShapeDtypeStruct((E, D), jnp.float32),
    )(upd, edge_feats, stats, gamma, beta)


def _node_out_body(gsum_ref, ndst_ref, nsrc_ref, nf_ref, gamma_ref, beta_ref,
                   o_ref):
    s = gsum_ref[0, :N, :] + gsum_ref[1, :N, :]
    n_gate = ndst_ref[...] * (s / (s + 1e-6))
    x = nsrc_ref[...] + n_gate
    mean = jnp.mean(x, axis=0, keepdims=True)
    var = jnp.mean((x - mean) ** 2, axis=0, keepdims=True)
    xh = (x - mean) * lax.rsqrt(var + 1e-5) * gamma_ref[...] + beta_ref[...]
    o_ref[...] = xh * jax.nn.sigmoid(xh) + nf_ref[...]


def _node_out(gsum, n_dst_lin, n_src_lin, node_feats, gamma, beta):
    return pl.pallas_call(
        _node_out_body,
        out_shape=jax.ShapeDtypeStruct((N, D), jnp.float32),
    )(gsum, n_dst_lin, n_src_lin, node_feats, gamma, beta)


def kernel(node_feats, edge_feats, W_esrc, b_esrc, W_edst, b_edst, W_eedge,
           b_eedge, W_nsrc, b_nsrc, W_ndst, b_ndst, gamma_e, beta_e, gamma_n,
           beta_n, edge_index):
    w_cat = jnp.concatenate([W_esrc, W_edst, W_ndst, W_nsrc], axis=1)
    b_cat = jnp.concatenate([b_esrc, b_edst, b_ndst, b_nsrc])[None, :]
    nodecat = _node_matmul(node_feats, w_cat, b_cat)
    e_src_t = nodecat[:, 0 * D:1 * D]
    e_dst_t = nodecat[:, 1 * D:2 * D]
    n_dst_lin = nodecat[:, 2 * D:3 * D]
    n_src_lin = nodecat[:, 3 * D:4 * D]

    edge_lin = _edge_matmul(edge_feats, W_eedge, b_eedge[None, :])

    src = edge_index[0]
    dst = edge_index[1]
    zeros = jnp.zeros((NPAD, D), jnp.float32)
    upd, gsum, stats = _sc_edge_kernel(
        edge_lin, src, dst, e_src_t, e_dst_t, zeros)

    edge_out = _edge_out(upd, edge_feats, stats, gamma_e[None, :],
                         beta_e[None, :])
    node_out = _node_out(gsum, n_dst_lin, n_src_lin, node_feats,
                         gamma_n[None, :], beta_n[None, :])
    return (node_out, edge_out)


# speedup vs baseline: 4.8629x; 1.3307x over previous
import functools

import jax
import jax.numpy as jnp
from jax import lax
from jax.experimental import pallas as pl
from jax.experimental.pallas import tpu as pltpu
from jax.experimental.pallas import tpu_sc as plsc

N = 10000
E = 320000
D = 128

NC = 2
NS = 16
NW = NC * NS
EPW = E // NW
C = 40
NCHUNK = EPW // C
RPT = 640
NLANE = D // 16


def _matmul_body(x_ref, w_ref, b_ref, o_ref):
    o_ref[...] = (
        jnp.dot(x_ref[...], w_ref[...], preferred_element_type=jnp.float32)
        + b_ref[...]
    )


def _node_matmul(node_feats, w_cat, b_cat):
    return pl.pallas_call(
        _matmul_body,
        out_shape=jax.ShapeDtypeStruct((N, 4 * D), jnp.float32),
    )(node_feats, w_cat, b_cat)


_EDGE_BLK = 1280


def _edge_matmul(edge_feats, w, b):
    grid = (E // _EDGE_BLK,)
    return pl.pallas_call(
        _matmul_body,
        grid=grid,
        in_specs=[
            pl.BlockSpec((_EDGE_BLK, D), lambda i: (i, 0)),
            pl.BlockSpec((D, D), lambda i: (0, 0)),
            pl.BlockSpec((1, D), lambda i: (0, 0)),
        ],
        out_specs=pl.BlockSpec((_EDGE_BLK, D), lambda i: (i, 0)),
        out_shape=jax.ShapeDtypeStruct((E, D), jnp.float32),
    )(edge_feats, w, b)


_SC_MESH = plsc.VectorSubcoreMesh(core_axis_name="c", subcore_axis_name="s")


_NBUF = 2
SC_K = 25
NSUP = NCHUNK // SC_K


@functools.partial(
    pl.kernel,
    out_type=(
        jax.ShapeDtypeStruct((E, D), jnp.float32),
        jax.ShapeDtypeStruct((NC, N, D), jnp.float32),
        jax.ShapeDtypeStruct((NW, 2, D), jnp.float32),
    ),
    mesh=_SC_MESH,
    scratch_types=[
        pltpu.VMEM((_NBUF * SC_K, 2 * C), jnp.int32),
        pltpu.VMEM((_NBUF * SC_K, C), jnp.int32),
        [pltpu.VMEM((2 * C, D), jnp.float32) for _ in range(_NBUF)],
        [pltpu.VMEM((C, D), jnp.float32) for _ in range(_NBUF)],
        pltpu.VMEM((2, D), jnp.float32),
        pltpu.VMEM_SHARED((N, D), jnp.float32),
        [pltpu.SemaphoreType.DMA for _ in range(_NBUF)],
        [pltpu.SemaphoreType.DMA for _ in range(_NBUF)],
        pltpu.SemaphoreType.DMA,
    ],
)
def _sc_edge_kernel(lin_hbm, idxcat_hbm, idst_hbm, tab_hbm, zeros_hbm,
                    upd_hbm, gsum_hbm, stats_hbm,
                    idxb, idstb, gbuf, linb, statsb, acc, sin, sout, sidx):
    cid = lax.axis_index("c")
    sid = lax.axis_index("s")
    wid = sid * NC + cid

    z16 = jnp.zeros((16,), jnp.float32)
    for j in range(NLANE):
        statsb[0, pl.ds(j * 16, 16)] = z16
        statsb[1, pl.ds(j * 16, 16)] = z16

    zbase = jnp.minimum(sid * RPT, N - RPT)
    pltpu.sync_copy(
        zeros_hbm.at[pl.ds(zbase, RPT), :],
        acc.at[pl.ds(zbase, RPT), :],
    )
    plsc.subcore_barrier()

    def issue_idx(t):
        for spv in range(_NBUF):
            @pl.when(t % _NBUF == spv)
            def _():
                pltpu.async_copy(idxcat_hbm.at[wid, t],
                                 idxb.at[pl.ds(spv * SC_K, SC_K), :], sidx)
                pltpu.async_copy(idst_hbm.at[wid, t],
                                 idstb.at[pl.ds(spv * SC_K, SC_K), :], sidx)

    def wait_idx():
        pltpu.make_async_copy(idxcat_hbm.at[0, 0],
                              idxb.at[pl.ds(0, SC_K), :], sidx).wait()
        pltpu.make_async_copy(idst_hbm.at[0, 0],
                              idstb.at[pl.ds(0, SC_K), :], sidx).wait()

    def _row(k):
        return (k // SC_K) % _NBUF * SC_K + k % SC_K

    def issue_in(k, q):
        base = wid * EPW + k * C
        pltpu.async_copy(tab_hbm.at[idxb.at[_row(k)]], gbuf[q], sin[q])
        pltpu.async_copy(lin_hbm.at[pl.ds(base, C), :], linb[q], sin[q])

    def wait_in(p):
        pltpu.make_async_copy(tab_hbm.at[idxb.at[0]], gbuf[p], sin[p]).wait()
        pltpu.make_async_copy(lin_hbm.at[pl.ds(0, C), :], linb[p], sin[p]).wait()

    def issue_out(k, p):
        base = wid * EPW + k * C
        pltpu.async_copy(linb[p], upd_hbm.at[pl.ds(base, C), :], sout[p])
        pltpu.sync_copy(gbuf[p].at[pl.ds(0, C), :], acc.at[idstb.at[_row(k)]],
                        add=True)

    def wait_out(q):
        pltpu.make_async_copy(linb[q], upd_hbm.at[pl.ds(0, C), :], sout[q]).wait()

    pltpu.sync_copy(idxcat_hbm.at[wid, 0], idxb.at[pl.ds(0, SC_K), :])
    pltpu.sync_copy(idst_hbm.at[wid, 0], idstb.at[pl.ds(0, SC_K), :])
    @pl.when(NSUP > 1)
    def _():
        issue_idx(1)
    issue_in(0, 0)

    def chunk_body(k, _):
        t = k // SC_K
        for p in range(_NBUF):
            q = (p + 1) % _NBUF

            @pl.when(k % _NBUF == p)
            def _():
                @pl.when(k >= 1)
                def _():
                    wait_out(q)

                @pl.when((k % SC_K == 0) & (k >= SC_K) & (t + 1 < NSUP))
                def _():
                    issue_idx(t + 1)

                @pl.when((k % SC_K == SC_K - 1) & (k + 1 < NCHUNK))
                def _():
                    wait_idx()

                @pl.when(k + 1 < NCHUNK)
                def _():
                    issue_in(k + 1, q)

                wait_in(p)

                def row_body(r, rc):
                    sums, sqs = rc
                    new_sums, new_sqs = [], []
                    for j in range(NLANE):
                        sl = pl.ds(j * 16, 16)
                        u = linb[p][r, sl] + gbuf[p][r, sl] + gbuf[p][r + C, sl]
                        linb[p][r, sl] = u
                        gbuf[p][r, sl] = u / (1.0 + jnp.exp(-u))
                        new_sums.append(sums[j] + u)
                        new_sqs.append(sqs[j] + u * u)
                    return (tuple(new_sums), tuple(new_sqs))

                init = (tuple(z16 for _ in range(NLANE)),
                        tuple(z16 for _ in range(NLANE)))
                sums, sqs = lax.fori_loop(0, C, row_body, init)
                for j in range(NLANE):
                    plsc.addupdate(statsb.at[0, pl.ds(j * 16, 16)], sums[j])
                    plsc.addupdate(statsb.at[1, pl.ds(j * 16, 16)], sqs[j])
                issue_out(k, p)
        return 0

    lax.fori_loop(0, NCHUNK, chunk_body, 0)
    wait_out((NCHUNK - 1) % _NBUF)

    pltpu.sync_copy(statsb, stats_hbm.at[wid])

    plsc.subcore_barrier()
    pltpu.sync_copy(
        acc.at[pl.ds(zbase, RPT), :],
        gsum_hbm.at[cid, pl.ds(zbase, RPT), :],
    )


def _edge_out_body(upd_ref, ef_ref, stats_ref, gamma_ref, beta_ref, o_ref):
    s = jnp.sum(stats_ref[...], axis=0)
    mean = s[0] / E
    var = s[1] / E - mean * mean
    rstd = lax.rsqrt(var + 1e-5)
    u = upd_ref[...]
    xh = (u - mean[None, :]) * rstd[None, :] * gamma_ref[...] + beta_ref[...]
    o_ref[...] = xh * jax.nn.sigmoid(xh) + ef_ref[...]


def _edge_out(upd, edge_feats, stats, gamma, beta):
    grid = (E // _EDGE_BLK,)
    return pl.pallas_call(
        _edge_out_body,
        grid=grid,
        in_specs=[
            pl.BlockSpec((_EDGE_BLK, D), lambda i: (i, 0)),
            pl.BlockSpec((_EDGE_BLK, D), lambda i: (i, 0)),
            pl.BlockSpec((NW, 2, D), lambda i: (0, 0, 0)),
            pl.BlockSpec((1, D), lambda i: (0, 0)),
            pl.BlockSpec((1, D), lambda i: (0, 0)),
        ],
        out_specs=pl.BlockSpec((_EDGE_BLK, D), lambda i: (i, 0)),
        out_shape=jax.ShapeDtypeStruct((E, D), jnp.float32),
    )(upd, edge_feats, stats, gamma, beta)


def _node_out_body(gsum_ref, ndst_ref, nsrc_ref, nf_ref, gamma_ref, beta_ref,
                   o_ref):
    s = gsum_ref[0, :N, :] + gsum_ref[1, :N, :]
    n_gate = ndst_ref[...] * (s / (s + 1e-6))
    x = nsrc_ref[...] + n_gate
    mean = jnp.mean(x, axis=0, keepdims=True)
    var = jnp.mean((x - mean) ** 2, axis=0, keepdims=True)
    xh = (x - mean) * lax.rsqrt(var + 1e-5) * gamma_ref[...] + beta_ref[...]
    o_ref[...] = xh * jax.nn.sigmoid(xh) + nf_ref[...]


def _node_out(gsum, n_dst_lin, n_src_lin, node_feats, gamma, beta):
    return pl.pallas_call(
        _node_out_body,
        out_shape=jax.ShapeDtypeStruct((N, D), jnp.float32),
    )(gsum, n_dst_lin, n_src_lin, node_feats, gamma, beta)


def kernel(node_feats, edge_feats, W_esrc, b_esrc, W_edst, b_edst, W_eedge,
           b_eedge, W_nsrc, b_nsrc, W_ndst, b_ndst, gamma_e, beta_e, gamma_n,
           beta_n, edge_index):
    w_cat = jnp.concatenate([W_esrc, W_edst, W_ndst, W_nsrc], axis=1)
    b_cat = jnp.concatenate([b_esrc, b_edst, b_ndst, b_nsrc])[None, :]
    nodecat = _node_matmul(node_feats, w_cat, b_cat)
    e_src_t = nodecat[:, 0 * D:1 * D]
    e_dst_t = nodecat[:, 1 * D:2 * D]
    n_dst_lin = nodecat[:, 2 * D:3 * D]
    n_src_lin = nodecat[:, 3 * D:4 * D]

    edge_lin = _edge_matmul(edge_feats, W_eedge, b_eedge[None, :])

    src = edge_index[0].reshape(NW, NCHUNK, C)
    dst = edge_index[1].reshape(NW, NCHUNK, C)
    idxcat = jnp.concatenate([src, dst + N], axis=-1)
    idxcat = idxcat.reshape(NW, NSUP, SC_K, 2 * C)
    idst = dst.reshape(NW, NSUP, SC_K, C)
    tab = jnp.concatenate([e_src_t, e_dst_t], axis=0)
    zeros = jnp.zeros((N, D), jnp.float32)
    upd, gsum, stats = _sc_edge_kernel(
        edge_lin, idxcat, idst, tab, zeros)

    edge_out = _edge_out(upd, edge_feats, stats, gamma_e[None, :],
                         beta_e[None, :])
    node_out = _node_out(gsum, n_dst_lin, n_src_lin, node_feats,
                         gamma_n[None, :], beta_n[None, :])
    return (node_out, edge_out)


# speedup vs baseline: 6.3093x; 1.2974x over previous
import functools

import jax
import jax.numpy as jnp
from jax import lax
from jax.experimental import pallas as pl
from jax.experimental.pallas import tpu as pltpu
from jax.experimental.pallas import tpu_sc as plsc

N = 10000
E = 320000
D = 128

NC = 2
NS = 16
NW = NC * NS
EPW = E // NW
C = 40
NCHUNK = EPW // C
RPT = 640
NLANE = D // 16


def _matmul_body(x_ref, w_ref, b_ref, o_ref):
    o_ref[...] = (
        jnp.dot(x_ref[...], w_ref[...], preferred_element_type=jnp.float32)
        + b_ref[...]
    )


_NODE_BLK = 2000


def _node_matmul_body(x_ref, w_ref, b_ref, o_ref):
    o_ref[0] = (
        jnp.dot(x_ref[...], w_ref[0], preferred_element_type=jnp.float32)
        + b_ref[0]
    )


def _node_matmul(node_feats, w_stack, b_stack):
    return pl.pallas_call(
        _node_matmul_body,
        grid=(4, N // _NODE_BLK),
        in_specs=[
            pl.BlockSpec((_NODE_BLK, D), lambda j, i: (i, 0)),
            pl.BlockSpec((1, D, D), lambda j, i: (j, 0, 0)),
            pl.BlockSpec((1, 1, D), lambda j, i: (j, 0, 0)),
        ],
        out_specs=pl.BlockSpec((1, _NODE_BLK, D), lambda j, i: (j, i, 0)),
        out_shape=jax.ShapeDtypeStruct((4, N, D), jnp.float32),
    )(node_feats, w_stack, b_stack)


_EDGE_BLK = 6400


def _edge_matmul(edge_feats, w, b):
    grid = (E // _EDGE_BLK,)
    return pl.pallas_call(
        _matmul_body,
        grid=grid,
        in_specs=[
            pl.BlockSpec((_EDGE_BLK, D), lambda i: (i, 0)),
            pl.BlockSpec((D, D), lambda i: (0, 0)),
            pl.BlockSpec((1, D), lambda i: (0, 0)),
        ],
        out_specs=pl.BlockSpec((_EDGE_BLK, D), lambda i: (i, 0)),
        out_shape=jax.ShapeDtypeStruct((E, D), jnp.float32),
    )(edge_feats, w, b)


_SC_MESH = plsc.VectorSubcoreMesh(core_axis_name="c", subcore_axis_name="s")


_NBUF = 2
SC_K = 25
NSUP = NCHUNK // SC_K


@functools.partial(
    pl.kernel,
    out_type=(
        jax.ShapeDtypeStruct((E, D), jnp.float32),
        jax.ShapeDtypeStruct((NC, N, D), jnp.float32),
        jax.ShapeDtypeStruct((NW, 2, D), jnp.float32),
    ),
    mesh=_SC_MESH,
    scratch_types=[
        pltpu.VMEM((_NBUF * SC_K, 2 * C), jnp.int32),
        pltpu.VMEM((_NBUF * SC_K, C), jnp.int32),
        [pltpu.VMEM((2 * C, D), jnp.float32) for _ in range(_NBUF)],
        [pltpu.VMEM((C, D), jnp.float32) for _ in range(_NBUF)],
        pltpu.VMEM((2, D), jnp.float32),
        pltpu.VMEM_SHARED((N, D), jnp.float32),
        [pltpu.SemaphoreType.DMA for _ in range(_NBUF)],
        [pltpu.SemaphoreType.DMA for _ in range(_NBUF)],
        pltpu.SemaphoreType.DMA,
    ],
)
def _sc_edge_kernel(lin_hbm, idxcat_hbm, idst_hbm, tab_hbm, zeros_hbm,
                    upd_hbm, gsum_hbm, stats_hbm,
                    idxb, idstb, gbuf, linb, statsb, acc, sin, sout, sidx):
    cid = lax.axis_index("c")
    sid = lax.axis_index("s")
    wid = sid * NC + cid

    z16 = jnp.zeros((16,), jnp.float32)
    for j in range(NLANE):
        statsb[0, pl.ds(j * 16, 16)] = z16
        statsb[1, pl.ds(j * 16, 16)] = z16

    zbase = jnp.minimum(sid * RPT, N - RPT)
    pltpu.sync_copy(
        zeros_hbm.at[pl.ds(zbase, RPT), :],
        acc.at[pl.ds(zbase, RPT), :],
    )
    plsc.subcore_barrier()

    def issue_idx(t):
        for spv in range(_NBUF):
            @pl.when(t % _NBUF == spv)
            def _():
                pltpu.async_copy(idxcat_hbm.at[wid, t],
                                 idxb.at[pl.ds(spv * SC_K, SC_K), :], sidx)
                pltpu.async_copy(idst_hbm.at[wid, t],
                                 idstb.at[pl.ds(spv * SC_K, SC_K), :], sidx)

    def wait_idx():
        pltpu.make_async_copy(idxcat_hbm.at[0, 0],
                              idxb.at[pl.ds(0, SC_K), :], sidx).wait()
        pltpu.make_async_copy(idst_hbm.at[0, 0],
                              idstb.at[pl.ds(0, SC_K), :], sidx).wait()

    def _row(k):
        return (k // SC_K) % _NBUF * SC_K + k % SC_K

    def issue_in(k, q):
        base = wid * EPW + k * C
        pltpu.async_copy(tab_hbm.at[idxb.at[_row(k)]], gbuf[q], sin[q])
        pltpu.async_copy(lin_hbm.at[pl.ds(base, C), :], linb[q], sin[q])

    def wait_in(p):
        pltpu.make_async_copy(tab_hbm.at[idxb.at[0]], gbuf[p], sin[p]).wait()
        pltpu.make_async_copy(lin_hbm.at[pl.ds(0, C), :], linb[p], sin[p]).wait()

    def issue_out(k, p):
        base = wid * EPW + k * C
        pltpu.async_copy(linb[p], upd_hbm.at[pl.ds(base, C), :], sout[p])
        pltpu.sync_copy(gbuf[p].at[pl.ds(0, C), :], acc.at[idstb.at[_row(k)]],
                        add=True)

    def wait_out(q):
        pltpu.make_async_copy(linb[q], upd_hbm.at[pl.ds(0, C), :], sout[q]).wait()

    pltpu.sync_copy(idxcat_hbm.at[wid, 0], idxb.at[pl.ds(0, SC_K), :])
    pltpu.sync_copy(idst_hbm.at[wid, 0], idstb.at[pl.ds(0, SC_K), :])
    @pl.when(NSUP > 1)
    def _():
        issue_idx(1)
    issue_in(0, 0)

    def chunk_body(k, _):
        t = k // SC_K
        for p in range(_NBUF):
            q = (p + 1) % _NBUF

            @pl.when(k % _NBUF == p)
            def _():
                @pl.when(k >= 1)
                def _():
                    wait_out(q)

                @pl.when((k % SC_K == 0) & (k >= SC_K) & (t + 1 < NSUP))
                def _():
                    issue_idx(t + 1)

                @pl.when((k % SC_K == SC_K - 1) & (k + 1 < NCHUNK))
                def _():
                    wait_idx()

                @pl.when(k + 1 < NCHUNK)
                def _():
                    issue_in(k + 1, q)

                wait_in(p)

                def row_body(r, rc):
                    sums, sqs = rc
                    new_sums, new_sqs = [], []
                    for j in range(NLANE):
                        sl = pl.ds(j * 16, 16)
                        u = linb[p][r, sl] + gbuf[p][r, sl] + gbuf[p][r + C, sl]
                        linb[p][r, sl] = u
                        gbuf[p][r, sl] = u / (1.0 + jnp.exp(-u))
                        new_sums.append(sums[j] + u)
                        new_sqs.append(sqs[j] + u * u)
                    return (tuple(new_sums), tuple(new_sqs))

                init = (tuple(z16 for _ in range(NLANE)),
                        tuple(z16 for _ in range(NLANE)))
                sums, sqs = lax.fori_loop(0, C, row_body, init)
                for j in range(NLANE):
                    plsc.addupdate(statsb.at[0, pl.ds(j * 16, 16)], sums[j])
                    plsc.addupdate(statsb.at[1, pl.ds(j * 16, 16)], sqs[j])
                issue_out(k, p)
        return 0

    lax.fori_loop(0, NCHUNK, chunk_body, 0)
    wait_out((NCHUNK - 1) % _NBUF)

    pltpu.sync_copy(statsb, stats_hbm.at[wid])

    plsc.subcore_barrier()
    pltpu.sync_copy(
        acc.at[pl.ds(zbase, RPT), :],
        gsum_hbm.at[cid, pl.ds(zbase, RPT), :],
    )


def _edge_out_body(upd_ref, ef_ref, stats_ref, gamma_ref, beta_ref, o_ref):
    s = jnp.sum(stats_ref[...], axis=0)
    mean = s[0] / E
    var = s[1] / E - mean * mean
    rstd = lax.rsqrt(var + 1e-5)
    u = upd_ref[...]
    xh = (u - mean[None, :]) * rstd[None, :] * gamma_ref[...] + beta_ref[...]
    o_ref[...] = xh * jax.nn.sigmoid(xh) + ef_ref[...]


def _edge_out(upd, edge_feats, stats, gamma, beta):
    grid = (E // _EDGE_BLK,)
    return pl.pallas_call(
        _edge_out_body,
        grid=grid,
        in_specs=[
            pl.BlockSpec((_EDGE_BLK, D), lambda i: (i, 0)),
            pl.BlockSpec((_EDGE_BLK, D), lambda i: (i, 0)),
            pl.BlockSpec((NW, 2, D), lambda i: (0, 0, 0)),
            pl.BlockSpec((1, D), lambda i: (0, 0)),
            pl.BlockSpec((1, D), lambda i: (0, 0)),
        ],
        out_specs=pl.BlockSpec((_EDGE_BLK, D), lambda i: (i, 0)),
        out_shape=jax.ShapeDtypeStruct((E, D), jnp.float32),
    )(upd, edge_feats, stats, gamma, beta)


def _node_out_body(gsum_ref, ndst_ref, nsrc_ref, nf_ref, gamma_ref, beta_ref,
                   o_ref):
    s = gsum_ref[0, :N, :] + gsum_ref[1, :N, :]
    n_gate = ndst_ref[...] * (s / (s + 1e-6))
    x = nsrc_ref[...] + n_gate
    mean = jnp.mean(x, axis=0, keepdims=True)
    var = jnp.mean((x - mean) ** 2, axis=0, keepdims=True)
    xh = (x - mean) * lax.rsqrt(var + 1e-5) * gamma_ref[...] + beta_ref[...]
    o_ref[...] = xh * jax.nn.sigmoid(xh) + nf_ref[...]


def _node_out(gsum, n_dst_lin, n_src_lin, node_feats, gamma, beta):
    return pl.pallas_call(
        _node_out_body,
        out_shape=jax.ShapeDtypeStruct((N, D), jnp.float32),
    )(gsum, n_dst_lin, n_src_lin, node_feats, gamma, beta)


def kernel(node_feats, edge_feats, W_esrc, b_esrc, W_edst, b_edst, W_eedge,
           b_eedge, W_nsrc, b_nsrc, W_ndst, b_ndst, gamma_e, beta_e, gamma_n,
           beta_n, edge_index):
    w_stack = jnp.stack([W_esrc, W_edst, W_ndst, W_nsrc])
    b_stack = jnp.stack([b_esrc, b_edst, b_ndst, b_nsrc])[:, None, :]
    nodecat = _node_matmul(node_feats, w_stack, b_stack)
    tab = nodecat[:2].reshape(2 * N, D)
    n_dst_lin = nodecat[2]
    n_src_lin = nodecat[3]

    edge_lin = _edge_matmul(edge_feats, W_eedge, b_eedge[None, :])

    src = edge_index[0].reshape(NW, NCHUNK, C)
    dst = edge_index[1].reshape(NW, NCHUNK, C)
    idxcat = jnp.concatenate([src, dst + N], axis=-1)
    idxcat = idxcat.reshape(NW, NSUP, SC_K, 2 * C)
    idst = dst.reshape(NW, NSUP, SC_K, C)
    zeros = jnp.zeros((N, D), jnp.float32)
    upd, gsum, stats = _sc_edge_kernel(
        edge_lin, idxcat, idst, tab, zeros)

    edge_out = _edge_out(upd, edge_feats, stats, gamma_e[None, :],
                         beta_e[None, :])
    node_out = _node_out(gsum, n_dst_lin, n_src_lin, node_feats,
                         gamma_n[None, :], beta_n[None, :])
    return (node_out, edge_out)


# speedup vs baseline: 6.3527x; 1.0069x over previous
import functools

import jax
import jax.numpy as jnp
from jax import lax
from jax.experimental import pallas as pl
from jax.experimental.pallas import tpu as pltpu
from jax.experimental.pallas import tpu_sc as plsc

N = 10000
E = 320000
D = 128

NC = 2
NS = 16
NW = NC * NS
EPW = E // NW
C = 40
NCHUNK = EPW // C
RPT = 640
NLANE = D // 16


def _matmul_body(x_ref, w_ref, b_ref, o_ref):
    o_ref[...] = (
        jnp.dot(x_ref[...], w_ref[...], preferred_element_type=jnp.float32)
        + b_ref[...]
    )


_NODE_BLK = 2000


def _node_matmul_body(x_ref, w_ref, b_ref, o_ref):
    o_ref[0] = (
        jnp.dot(x_ref[...], w_ref[0], preferred_element_type=jnp.float32)
        + b_ref[0]
    )


def _node_matmul(node_feats, w_stack, b_stack):
    return pl.pallas_call(
        _node_matmul_body,
        grid=(4, N // _NODE_BLK),
        in_specs=[
            pl.BlockSpec((_NODE_BLK, D), lambda j, i: (i, 0)),
            pl.BlockSpec((1, D, D), lambda j, i: (j, 0, 0)),
            pl.BlockSpec((1, 1, D), lambda j, i: (j, 0, 0)),
        ],
        out_specs=pl.BlockSpec((1, _NODE_BLK, D), lambda j, i: (j, i, 0)),
        out_shape=jax.ShapeDtypeStruct((4, N, D), jnp.float32),
    )(node_feats, w_stack, b_stack)


_EDGE_BLK = 8000


def _edge_matmul(edge_feats, w, b):
    grid = (E // _EDGE_BLK,)
    return pl.pallas_call(
        _matmul_body,
        grid=grid,
        in_specs=[
            pl.BlockSpec((_EDGE_BLK, D), lambda i: (i, 0)),
            pl.BlockSpec((D, D), lambda i: (0, 0)),
            pl.BlockSpec((1, D), lambda i: (0, 0)),
        ],
        out_specs=pl.BlockSpec((_EDGE_BLK, D), lambda i: (i, 0)),
        out_shape=jax.ShapeDtypeStruct((E, D), jnp.float32),
    )(edge_feats, w, b)


_SC_MESH = plsc.VectorSubcoreMesh(core_axis_name="c", subcore_axis_name="s")


_NBUF = 2
SC_K = 25
NSUP = NCHUNK // SC_K


@functools.partial(
    pl.kernel,
    out_type=(
        jax.ShapeDtypeStruct((E, D), jnp.float32),
        jax.ShapeDtypeStruct((NC, N, D), jnp.float32),
        jax.ShapeDtypeStruct((NW, 2, D), jnp.float32),
    ),
    mesh=_SC_MESH,
    scratch_types=[
        pltpu.VMEM((_NBUF * SC_K, 2 * C), jnp.int32),
        pltpu.VMEM((_NBUF * SC_K, C), jnp.int32),
        [pltpu.VMEM((2 * C, D), jnp.float32) for _ in range(_NBUF)],
        [pltpu.VMEM((C, D), jnp.float32) for _ in range(_NBUF)],
        pltpu.VMEM((2, D), jnp.float32),
        pltpu.VMEM_SHARED((N, D), jnp.float32),
        [pltpu.SemaphoreType.DMA for _ in range(_NBUF)],
        [pltpu.SemaphoreType.DMA for _ in range(_NBUF)],
        pltpu.SemaphoreType.DMA,
    ],
)
def _sc_edge_kernel(lin_hbm, idxcat_hbm, idst_hbm, tab_hbm, zeros_hbm,
                    upd_hbm, gsum_hbm, stats_hbm,
                    idxb, idstb, gbuf, linb, statsb, acc, sin, sout, sidx):
    cid = lax.axis_index("c")
    sid = lax.axis_index("s")
    wid = sid * NC + cid

    z16 = jnp.zeros((16,), jnp.float32)
    for j in range(NLANE):
        statsb[0, pl.ds(j * 16, 16)] = z16
        statsb[1, pl.ds(j * 16, 16)] = z16

    zbase = jnp.minimum(sid * RPT, N - RPT)
    pltpu.sync_copy(
        zeros_hbm.at[pl.ds(zbase, RPT), :],
        acc.at[pl.ds(zbase, RPT), :],
    )
    plsc.subcore_barrier()

    def issue_idx(t):
        for spv in range(_NBUF):
            @pl.when(t % _NBUF == spv)
            def _():
                pltpu.async_copy(idxcat_hbm.at[wid, t],
                                 idxb.at[pl.ds(spv * SC_K, SC_K), :], sidx)
                pltpu.async_copy(idst_hbm.at[wid, t],
                                 idstb.at[pl.ds(spv * SC_K, SC_K), :], sidx)

    def wait_idx():
        pltpu.make_async_copy(idxcat_hbm.at[0, 0],
                              idxb.at[pl.ds(0, SC_K), :], sidx).wait()
        pltpu.make_async_copy(idst_hbm.at[0, 0],
                              idstb.at[pl.ds(0, SC_K), :], sidx).wait()

    def _row(k):
        return (k // SC_K) % _NBUF * SC_K + k % SC_K

    def issue_in(k, q):
        base = wid * EPW + k * C
        pltpu.async_copy(tab_hbm.at[idxb.at[_row(k)]], gbuf[q], sin[q])
        pltpu.async_copy(lin_hbm.at[pl.ds(base, C), :], linb[q], sin[q])

    def wait_in(p):
        pltpu.make_async_copy(tab_hbm.at[idxb.at[0]], gbuf[p], sin[p]).wait()
        pltpu.make_async_copy(lin_hbm.at[pl.ds(0, C), :], linb[p], sin[p]).wait()

    def issue_out(k, p):
        base = wid * EPW + k * C
        pltpu.async_copy(linb[p], upd_hbm.at[pl.ds(base, C), :], sout[p])
        pltpu.sync_copy(gbuf[p].at[pl.ds(0, C), :], acc.at[idstb.at[_row(k)]],
                        add=True)

    def wait_out(q):
        pltpu.make_async_copy(linb[q], upd_hbm.at[pl.ds(0, C), :], sout[q]).wait()

    pltpu.sync_copy(idxcat_hbm.at[wid, 0], idxb.at[pl.ds(0, SC_K), :])
    pltpu.sync_copy(idst_hbm.at[wid, 0], idstb.at[pl.ds(0, SC_K), :])
    @pl.when(NSUP > 1)
    def _():
        issue_idx(1)
    issue_in(0, 0)

    def chunk_body(k, _):
        t = k // SC_K
        for p in range(_NBUF):
            q = (p + 1) % _NBUF

            @pl.when(k % _NBUF == p)
            def _():
                @pl.when(k >= 1)
                def _():
                    wait_out(q)

                @pl.when((k % SC_K == 0) & (k >= SC_K) & (t + 1 < NSUP))
                def _():
                    issue_idx(t + 1)

                @pl.when((k % SC_K == SC_K - 1) & (k + 1 < NCHUNK))
                def _():
                    wait_idx()

                @pl.when(k + 1 < NCHUNK)
                def _():
                    issue_in(k + 1, q)

                wait_in(p)

                def row_body(r, rc):
                    sums, sqs = rc
                    new_sums, new_sqs = [], []
                    for j in range(NLANE):
                        sl = pl.ds(j * 16, 16)
                        u = linb[p][r, sl] + gbuf[p][r, sl] + gbuf[p][r + C, sl]
                        linb[p][r, sl] = u
                        gbuf[p][r, sl] = u / (1.0 + jnp.exp(-u))
                        new_sums.append(sums[j] + u)
                        new_sqs.append(sqs[j] + u * u)
                    return (tuple(new_sums), tuple(new_sqs))

                init = (tuple(z16 for _ in range(NLANE)),
                        tuple(z16 for _ in range(NLANE)))
                sums, sqs = lax.fori_loop(0, C, row_body, init)
                for j in range(NLANE):
                    plsc.addupdate(statsb.at[0, pl.ds(j * 16, 16)], sums[j])
                    plsc.addupdate(statsb.at[1, pl.ds(j * 16, 16)], sqs[j])
                issue_out(k, p)
        return 0

    lax.fori_loop(0, NCHUNK, chunk_body, 0)
    wait_out((NCHUNK - 1) % _NBUF)

    pltpu.sync_copy(statsb, stats_hbm.at[wid])

    plsc.subcore_barrier()
    pltpu.sync_copy(
        acc.at[pl.ds(zbase, RPT), :],
        gsum_hbm.at[cid, pl.ds(zbase, RPT), :],
    )


def _edge_out_body(upd_ref, ef_ref, stats_ref, gamma_ref, beta_ref, o_ref):
    s = jnp.sum(stats_ref[...], axis=0)
    mean = s[0] / E
    var = s[1] / E - mean * mean
    rstd = lax.rsqrt(var + 1e-5)
    u = upd_ref[...]
    xh = (u - mean[None, :]) * rstd[None, :] * gamma_ref[...] + beta_ref[...]
    o_ref[...] = xh * jax.nn.sigmoid(xh) + ef_ref[...]


def _edge_out(upd, edge_feats, stats, gamma, beta):
    grid = (E // _EDGE_BLK,)
    return pl.pallas_call(
        _edge_out_body,
        grid=grid,
        in_specs=[
            pl.BlockSpec((_EDGE_BLK, D), lambda i: (i, 0)),
            pl.BlockSpec((_EDGE_BLK, D), lambda i: (i, 0)),
            pl.BlockSpec((NW, 2, D), lambda i: (0, 0, 0)),
            pl.BlockSpec((1, D), lambda i: (0, 0)),
            pl.BlockSpec((1, D), lambda i: (0, 0)),
        ],
        out_specs=pl.BlockSpec((_EDGE_BLK, D), lambda i: (i, 0)),
        out_shape=jax.ShapeDtypeStruct((E, D), jnp.float32),
    )(upd, edge_feats, stats, gamma, beta)


def _node_out_body(gsum_ref, ndst_ref, nsrc_ref, nf_ref, gamma_ref, beta_ref,
                   o_ref):
    s = gsum_ref[0, :N, :] + gsum_ref[1, :N, :]
    n_gate = ndst_ref[...] * (s / (s + 1e-6))
    x = nsrc_ref[...] + n_gate
    mean = jnp.mean(x, axis=0, keepdims=True)
    var = jnp.mean((x - mean) ** 2, axis=0, keepdims=True)
    xh = (x - mean) * lax.rsqrt(var + 1e-5) * gamma_ref[...] + beta_ref[...]
    o_ref[...] = xh * jax.nn.sigmoid(xh) + nf_ref[...]


def _node_out(gsum, n_dst_lin, n_src_lin, node_feats, gamma, beta):
    return pl.pallas_call(
        _node_out_body,
        out_shape=jax.ShapeDtypeStruct((N, D), jnp.float32),
    )(gsum, n_dst_lin, n_src_lin, node_feats, gamma, beta)


def kernel(node_feats, edge_feats, W_esrc, b_esrc, W_edst, b_edst, W_eedge,
           b_eedge, W_nsrc, b_nsrc, W_ndst, b_ndst, gamma_e, beta_e, gamma_n,
           beta_n, edge_index):
    w_stack = jnp.stack([W_esrc, W_edst, W_ndst, W_nsrc])
    b_stack = jnp.stack([b_esrc, b_edst, b_ndst, b_nsrc])[:, None, :]
    nodecat = _node_matmul(node_feats, w_stack, b_stack)
    tab = nodecat[:2].reshape(2 * N, D)
    n_dst_lin = nodecat[2]
    n_src_lin = nodecat[3]

    edge_lin = _edge_matmul(edge_feats, W_eedge, b_eedge[None, :])

    src = edge_index[0].reshape(NW, NCHUNK, C)
    dst = edge_index[1].reshape(NW, NCHUNK, C)
    idxcat = jnp.concatenate([src, dst + N], axis=-1)
    idxcat = idxcat.reshape(NW, NSUP, SC_K, 2 * C)
    idst = dst.reshape(NW, NSUP, SC_K, C)
    zeros = jnp.zeros((N, D), jnp.float32)
    upd, gsum, stats = _sc_edge_kernel(
        edge_lin, idxcat, idst, tab, zeros)

    edge_out = _edge_out(upd, edge_feats, stats, gamma_e[None, :],
                         beta_e[None, :])
    node_out = _node_out(gsum, n_dst_lin, n_src_lin, node_feats,
                         gamma_n[None, :], beta_n[None, :])
    return (node_out, edge_out)


# speedup vs baseline: 6.4561x; 1.0163x over previous
import functools

import jax
import jax.numpy as jnp
from jax import lax
from jax.experimental import pallas as pl
from jax.experimental.pallas import tpu as pltpu
from jax.experimental.pallas import tpu_sc as plsc

N = 10000
E = 320000
D = 128

NC = 2
NS = 16
NW = NC * NS
EPW = E // NW
C = 40
NCHUNK = EPW // C
RPT = 640
NLANE = D // 16

_NBUF = 2
SC_K = 25
NSUP = NCHUNK // SC_K

_EDGE_BLK = 8000
_NEDGE = E // _EDGE_BLK
_NSTEP = _NEDGE + 4


def _mm_body(ef_ref, we_ref, be_ref, nf_ref, ws_ref, bs_ref,
             lin_ref, node_ref):
    i = pl.program_id(0)
    lin_ref[...] = (
        jnp.dot(ef_ref[...], we_ref[...], preferred_element_type=jnp.float32)
        + be_ref[...]
    )

    @pl.when(i < 4)
    def _():
        node_ref[0] = (
            jnp.dot(nf_ref[...], ws_ref[0], preferred_element_type=jnp.float32)
            + bs_ref[0]
        )


def _matmuls(edge_feats, w_e, b_e, node_feats, w_stack, b_stack):
    return pl.pallas_call(
        _mm_body,
        grid=(_NSTEP,),
        in_specs=[
            pl.BlockSpec((_EDGE_BLK, D), lambda i: (jnp.maximum(i - 4, 0), 0)),
            pl.BlockSpec((D, D), lambda i: (0, 0)),
            pl.BlockSpec((1, D), lambda i: (0, 0)),
            pl.BlockSpec((N, D), lambda i: (0, 0)),
            pl.BlockSpec((1, D, D), lambda i: (jnp.minimum(i, 3), 0, 0)),
            pl.BlockSpec((1, 1, D), lambda i: (jnp.minimum(i, 3), 0, 0)),
        ],
        out_specs=[
            pl.BlockSpec((_EDGE_BLK, D), lambda i: (jnp.maximum(i - 4, 0), 0)),
            pl.BlockSpec((1, N, D), lambda i: (jnp.minimum(i, 3), 0, 0)),
        ],
        out_shape=[
            jax.ShapeDtypeStruct((E, D), jnp.float32),
            jax.ShapeDtypeStruct((4, N, D), jnp.float32),
        ],
    )(edge_feats, w_e, b_e, node_feats, w_stack, b_stack)


_SC_MESH = plsc.VectorSubcoreMesh(core_axis_name="c", subcore_axis_name="s")


@functools.partial(
    pl.kernel,
    out_type=(
        jax.ShapeDtypeStruct((E, D), jnp.float32),
        jax.ShapeDtypeStruct((NC, N, D), jnp.float32),
        jax.ShapeDtypeStruct((NW, 2, D), jnp.float32),
    ),
    mesh=_SC_MESH,
    scratch_types=[
        pltpu.VMEM((_NBUF * SC_K, 2 * C), jnp.int32),
        pltpu.VMEM((_NBUF * SC_K, C), jnp.int32),
        [pltpu.VMEM((2 * C, D), jnp.float32) for _ in range(_NBUF)],
        [pltpu.VMEM((C, D), jnp.float32) for _ in range(_NBUF)],
        pltpu.VMEM((2, D), jnp.float32),
        pltpu.VMEM_SHARED((N, D), jnp.float32),
        [pltpu.SemaphoreType.DMA for _ in range(_NBUF)],
        [pltpu.SemaphoreType.DMA for _ in range(_NBUF)],
        pltpu.SemaphoreType.DMA,
    ],
)
def _sc_edge_kernel(lin_hbm, idxcat_hbm, idst_hbm, tab_hbm, zeros_hbm,
                    upd_hbm, gsum_hbm, stats_hbm,
                    idxb, idstb, gbuf, linb, statsb, acc, sin, sout, sidx):
    cid = lax.axis_index("c")
    sid = lax.axis_index("s")
    wid = sid * NC + cid

    z16 = jnp.zeros((16,), jnp.float32)
    for j in range(NLANE):
        statsb[0, pl.ds(j * 16, 16)] = z16
        statsb[1, pl.ds(j * 16, 16)] = z16

    zbase = jnp.minimum(sid * RPT, N - RPT)
    pltpu.sync_copy(
        zeros_hbm.at[pl.ds(zbase, RPT), :],
        acc.at[pl.ds(zbase, RPT), :],
    )
    plsc.subcore_barrier()

    def issue_idx(t):
        for spv in range(_NBUF):
            @pl.when(t % _NBUF == spv)
            def _():
                pltpu.async_copy(idxcat_hbm.at[wid, t],
                                 idxb.at[pl.ds(spv * SC_K, SC_K), :], sidx)
                pltpu.async_copy(idst_hbm.at[wid, t],
                                 idstb.at[pl.ds(spv * SC_K, SC_K), :], sidx)

    def wait_idx():
        pltpu.make_async_copy(idxcat_hbm.at[0, 0],
                              idxb.at[pl.ds(0, SC_K), :], sidx).wait()
        pltpu.make_async_copy(idst_hbm.at[0, 0],
                              idstb.at[pl.ds(0, SC_K), :], sidx).wait()

    def _row(k):
        return (k // SC_K) % _NBUF * SC_K + k % SC_K

    def issue_in(k, q):
        base = wid * EPW + k * C
        pltpu.async_copy(tab_hbm.at[idxb.at[_row(k)]], gbuf[q], sin[q])
        pltpu.async_copy(lin_hbm.at[pl.ds(base, C), :], linb[q], sin[q])

    def wait_in(p):
        pltpu.make_async_copy(tab_hbm.at[idxb.at[0]], gbuf[p], sin[p]).wait()
        pltpu.make_async_copy(lin_hbm.at[pl.ds(0, C), :], linb[p], sin[p]).wait()

    def issue_out(k, p):
        base = wid * EPW + k * C
        pltpu.async_copy(linb[p], upd_hbm.at[pl.ds(base, C), :], sout[p])
        pltpu.sync_copy(gbuf[p].at[pl.ds(0, C), :], acc.at[idstb.at[_row(k)]],
                        add=True)

    def wait_out(q):
        pltpu.make_async_copy(linb[q], upd_hbm.at[pl.ds(0, C), :], sout[q]).wait()

    pltpu.sync_copy(idxcat_hbm.at[wid, 0], idxb.at[pl.ds(0, SC_K), :])
    pltpu.sync_copy(idst_hbm.at[wid, 0], idstb.at[pl.ds(0, SC_K), :])
    @pl.when(NSUP > 1)
    def _():
        issue_idx(1)
    issue_in(0, 0)

    def chunk_body(k, _):
        t = k // SC_K
        for p in range(_NBUF):
            q = (p + 1) % _NBUF

            @pl.when(k % _NBUF == p)
            def _():
                @pl.when(k >= 1)
                def _():
                    wait_out(q)

                @pl.when((k % SC_K == 0) & (k >= SC_K) & (t + 1 < NSUP))
                def _():
                    issue_idx(t + 1)

                @pl.when((k % SC_K == SC_K - 1) & (k + 1 < NCHUNK))
                def _():
                    wait_idx()

                @pl.when(k + 1 < NCHUNK)
                def _():
                    issue_in(k + 1, q)

                wait_in(p)

                def row_body(r, rc):
                    sums, sqs = rc
                    new_sums, new_sqs = [], []
                    for j in range(NLANE):
                        sl = pl.ds(j * 16, 16)
                        u = linb[p][r, sl] + gbuf[p][r, sl] + gbuf[p][r + C, sl]
                        linb[p][r, sl] = u
                        gbuf[p][r, sl] = u / (1.0 + jnp.exp(-u))
                        new_sums.append(sums[j] + u)
                        new_sqs.append(sqs[j] + u * u)
                    return (tuple(new_sums), tuple(new_sqs))

                init = (tuple(z16 for _ in range(NLANE)),
                        tuple(z16 for _ in range(NLANE)))
                sums, sqs = lax.fori_loop(0, C, row_body, init)
                for j in range(NLANE):
                    plsc.addupdate(statsb.at[0, pl.ds(j * 16, 16)], sums[j])
                    plsc.addupdate(statsb.at[1, pl.ds(j * 16, 16)], sqs[j])
                issue_out(k, p)
        return 0

    lax.fori_loop(0, NCHUNK, chunk_body, 0)
    wait_out((NCHUNK - 1) % _NBUF)

    pltpu.sync_copy(statsb, stats_hbm.at[wid])

    plsc.subcore_barrier()
    pltpu.sync_copy(
        acc.at[pl.ds(zbase, RPT), :],
        gsum_hbm.at[cid, pl.ds(zbase, RPT), :],
    )


def _apply_body(upd_ref, ef_ref, stats_ref, ge_ref, be_ref,
                gsum_ref, ndst_ref, nsrc_ref, nf_ref, gn_ref, bn_ref,
                eo_ref, no_ref):
    i = pl.program_id(0)
    s = jnp.sum(stats_ref[...], axis=0)
    mean = s[0] / E
    var = s[1] / E - mean * mean
    rstd = lax.rsqrt(var + 1e-5)
    u = upd_ref[...]
    xh = (u - mean[None, :]) * rstd[None, :] * ge_ref[...] + be_ref[...]
    eo_ref[...] = xh * jax.nn.sigmoid(xh) + ef_ref[...]

    @pl.when(i == _NEDGE)
    def _():
        sg = gsum_ref[0] + gsum_ref[1]
        n_gate = ndst_ref[...] * (sg / (sg + 1e-6))
        x = nsrc_ref[...] + n_gate
        nmean = jnp.mean(x, axis=0, keepdims=True)
        nvar = jnp.mean((x - nmean) ** 2, axis=0, keepdims=True)
        nxh = ((x - nmean) * lax.rsqrt(nvar + 1e-5) * gn_ref[...]
               + bn_ref[...])
        no_ref[...] = nxh * jax.nn.sigmoid(nxh) + nf_ref[...]


def _apply(upd, edge_feats, stats, gamma_e, beta_e,
           gsum, n_dst_lin, n_src_lin, node_feats, gamma_n, beta_n):
    eb = lambda i: (jnp.minimum(i, _NEDGE - 1), 0)
    c2 = lambda i: (0, 0)
    c3 = lambda i: (0, 0, 0)
    return pl.pallas_call(
        _apply_body,
        grid=(_NEDGE + 1,),
        in_specs=[
            pl.BlockSpec((_EDGE_BLK, D), eb),
            pl.BlockSpec((_EDGE_BLK, D), eb),
            pl.BlockSpec((NW, 2, D), c3),
            pl.BlockSpec((1, D), c2),
            pl.BlockSpec((1, D), c2),
            pl.BlockSpec((NC, N, D), c3),
            pl.BlockSpec((N, D), c2),
            pl.BlockSpec((N, D), c2),
            pl.BlockSpec((N, D), c2),
            pl.BlockSpec((1, D), c2),
            pl.BlockSpec((1, D), c2),
        ],
        out_specs=[
            pl.BlockSpec((_EDGE_BLK, D), eb),
            pl.BlockSpec((N, D), c2),
        ],
        out_shape=[
            jax.ShapeDtypeStruct((E, D), jnp.float32),
            jax.ShapeDtypeStruct((N, D), jnp.float32),
        ],
    )(upd, edge_feats, stats, gamma_e, beta_e,
      gsum, n_dst_lin, n_src_lin, node_feats, gamma_n, beta_n)


def kernel(node_feats, edge_feats, W_esrc, b_esrc, W_edst, b_edst, W_eedge,
           b_eedge, W_nsrc, b_nsrc, W_ndst, b_ndst, gamma_e, beta_e, gamma_n,
           beta_n, edge_index):
    w_stack = jnp.stack([W_esrc, W_edst, W_ndst, W_nsrc])
    b_stack = jnp.stack([b_esrc, b_edst, b_ndst, b_nsrc])[:, None, :]
    edge_lin, nodecat = _matmuls(edge_feats, W_eedge, b_eedge[None, :],
                                 node_feats, w_stack, b_stack)
    tab = nodecat[:2].reshape(2 * N, D)
    n_dst_lin = nodecat[2]
    n_src_lin = nodecat[3]

    src = edge_index[0].reshape(NW, NCHUNK, C)
    dst = edge_index[1].reshape(NW, NCHUNK, C)
    idxcat = jnp.concatenate([src, dst + N], axis=-1)
    idxcat = idxcat.reshape(NW, NSUP, SC_K, 2 * C)
    idst = dst.reshape(NW, NSUP, SC_K, C)
    zeros = jnp.zeros((N, D), jnp.float32)
    upd, gsum, stats = _sc_edge_kernel(
        edge_lin, idxcat, idst, tab, zeros)

    edge_out, node_out = _apply(
        upd, edge_feats, stats, gamma_e[None, :], beta_e[None, :],
        gsum, n_dst_lin, n_src_lin, node_feats,
        gamma_n[None, :], beta_n[None, :])
    return (node_out, edge_out)
